# baseline (speedup 1.0000x reference)
import functools

import jax
import jax.numpy as jnp
from jax import lax
from jax.experimental import pallas as pl
from jax.experimental.pallas import tpu as pltpu
from jax.experimental.pallas import tpu_sc as plsc

N = 10000
E = 160000
D = 128
NHEAD = 8

NC = 2
NS = 16
LANES = 16
NW = NC * NS
EV = E // NW
STRIP = 5120
NBATCH = STRIP // LANES
GROUP = 8
NGROUP = NBATCH // GROUP
GE = GROUP * LANES
E_PAD = NW * STRIP
NEG_BIG = -3.0e38

_mesh = lambda: plsc.VectorSubcoreMesh(core_axis_name="c", subcore_axis_name="s")
_params = lambda: pltpu.CompilerParams(use_tc_tiling_on_sc=False,
                                       needs_layout_passes=False)


def _wid():
  return lax.axis_index("s") * NC + lax.axis_index("c")


def _splat(vec, j):
  idx = jnp.full((LANES,), j, dtype=jnp.int32)
  return vec.at[idx].get(mode="promise_in_bounds")


GROUP_F = 4
GEF = GROUP_F * LANES
NGROUP_F = NBATCH // GROUP_F


def _make_fused(Hh, dh):
  HD = Hh * dh
  NROW = N // NS

  @functools.partial(
      pl.kernel,
      out_type=(jax.ShapeDtypeStruct((NW, Hh * STRIP), jnp.float32),
                jax.ShapeDtypeStruct((NC, Hh, N, dh), jnp.float32)),
      mesh=_mesh(),
      compiler_params=_params(),
      scratch_types=[
          pltpu.VMEM((STRIP,), jnp.int32),
          pltpu.VMEM((STRIP,), jnp.int32),
          pltpu.VMEM((STRIP,), jnp.float32),
          pltpu.VMEM((dh,), jnp.float32),
          pltpu.VMEM((GEF, dh), jnp.float32),
          pltpu.VMEM((GEF, dh), jnp.float32),
          pltpu.VMEM((GEF, dh), jnp.float32),
          pltpu.VMEM((GEF, dh), jnp.float32),
          pltpu.VMEM((GEF,), jnp.int32),
          pltpu.VMEM((GEF,), jnp.int32),
          pltpu.VMEM((GEF,), jnp.int32),
          pltpu.VMEM((GEF,), jnp.int32),
          pltpu.VMEM((125, LANES), jnp.float32),
          pltpu.VMEM_SHARED((N, dh), jnp.float32),
          pltpu.SemaphoreType.DMA,
          pltpu.SemaphoreType.DMA,
          pltpu.SemaphoreType.DMA,
          pltpu.SemaphoreType.DMA,
      ],
  )
  def k(feat_hbm, src_hbm, dst_hbm, attn_hbm, e_hbm, out_hbm,
        src_v, dst_v, e_h, attn_h, fs0, ft0, fs1, ft1,
        gx0, fx0, gx1, fx1, zb, acc_sp, semg0, semg1, sems0, sems1):
    cc = lax.axis_index("c")
    sid = lax.axis_index("s")
    wid = sid * NC + cc
    base = wid * STRIP
    rowb = jnp.arange(LANES, dtype=jnp.int32)
    pltpu.sync_copy(src_hbm.at[pl.ds(base, STRIP)], src_v)
    pltpu.sync_copy(dst_hbm.at[pl.ds(base, STRIP)], dst_v)
    zv = jnp.zeros((LANES,), jnp.float32)
    for r in range(125):
      zb[r, :] = zv

    def issue(hh, g, fs, ft, gx, fx, semg):
      goff = g * GEF
      for bb in range(GROUP_F):
        sl = pl.ds(bb * LANES, LANES)
        gsl = pl.ds(goff + bb * LANES, LANES)
        gx[sl] = src_v[gsl] + hh * N
        fx[sl] = dst_v[gsl] + hh * N
      pltpu.async_copy(feat_hbm.at[gx], fs, semg)
      pltpu.async_copy(feat_hbm.at[fx], ft, semg)

    def wait_g(fs, ft, gx, fx, semg):
      pltpu.make_async_copy(feat_hbm.at[gx], fs, semg).wait()
      pltpu.make_async_copy(feat_hbm.at[fx], ft, semg).wait()

    def process(hh, g, fs, ft, gx, sems):
      goff = g * GEF
      for bb in range(GROUP_F):
        rowv = bb * LANES + rowb
        def chunk(c2, acc):
          dbase = c2 * LANES
          a_chunk = attn_h[pl.ds(dbase, LANES)]
          for j in range(LANES):
            colv = jnp.full((LANES,), dbase + j, dtype=jnp.int32)
            z = (plsc.load_gather(fs, [rowv, colv]) +
                 plsc.load_gather(ft, [rowv, colv]))
            z = jnp.maximum(z, 0.2 * z)
            acc = acc + z * _splat(a_chunk, j)
          return acc
        acc = lax.fori_loop(0, dh // LANES, chunk,
                            jnp.zeros((LANES,), jnp.float32))
        boff = goff + bb * LANES
        e_h[pl.ds(boff, LANES)] = acc
        ee = jnp.exp(acc)
        ee = jnp.where(boff + rowb < EV, ee, 0.0)
        for j in range(LANES):
          asp = _splat(ee, j)
          r = bb * LANES + j
          for c2 in range(dh // LANES):
            sl = pl.ds(c2 * LANES, LANES)
            fs[r, sl] = fs[r, sl] * asp
        gx[pl.ds(bb * LANES, LANES)] = dst_v[pl.ds(boff, LANES)]
      pltpu.async_copy(fs, acc_sp.at[gx], sems, add=True)

    def head(hh, _):
      pltpu.sync_copy(attn_hbm.at[pl.ds(hh * dh, dh)], attn_h)
      def zloop(t, _z):
        r0 = sid * NROW + (t // (dh // LANES)) * 125
        c0 = (t % (dh // LANES)) * LANES
        pltpu.sync_copy(zb, acc_sp.at[pl.ds(r0, 125), pl.ds(c0, LANES)])
        return 0
      lax.fori_loop(0, (NROW // 125) * (dh // LANES), zloop, 0)
      plsc.subcore_barrier()

      issue(hh, 0, fs0, ft0, gx0, fx0, semg0)
      issue(hh, 1, fs1, ft1, gx1, fx1, semg1)

      def outer(i, _o):
        g0 = 2 * i
        wait_g(fs0, ft0, gx0, fx0, semg0)
        process(hh, g0, fs0, ft0, gx0, sems0)
        g1 = g0 + 1
        wait_g(fs1, ft1, gx1, fx1, semg1)
        process(hh, g1, fs1, ft1, gx1, sems1)
        pltpu.make_async_copy(fs0, acc_sp.at[gx0], sems0).wait()
        @pl.when(g0 + 2 < NGROUP_F)
        def _():
          issue(hh, g0 + 2, fs0, ft0, gx0, fx0, semg0)
        pltpu.make_async_copy(fs1, acc_sp.at[gx1], sems1).wait()
        @pl.when(g1 + 2 < NGROUP_F)
        def _():
          issue(hh, g1 + 2, fs1, ft1, gx1, fx1, semg1)
        return 0
      lax.fori_loop(0, NGROUP_F // 2, outer, 0)
      pltpu.sync_copy(e_h, e_hbm.at[wid, pl.ds(hh * STRIP, STRIP)])
      plsc.subcore_barrier()
      pltpu.sync_copy(acc_sp.at[pl.ds(sid * NROW, NROW)],
                      out_hbm.at[cc, hh, pl.ds(sid * NROW, NROW)])
      plsc.subcore_barrier()
      return 0
    lax.fori_loop(0, Hh, head, 0)

  return k


def _make_denom(Hh):
  NROW = N // NS

  @functools.partial(
      pl.kernel,
      out_type=jax.ShapeDtypeStruct((NC, N, LANES), jnp.float32),
      mesh=_mesh(),
      compiler_params=_params(),
      scratch_types=[
          pltpu.VMEM((Hh * STRIP,), jnp.float32),
          pltpu.VMEM((STRIP,), jnp.int32),
          pltpu.VMEM((GE, LANES), jnp.float32),
          pltpu.VMEM((GE,), jnp.int32),
          pltpu.VMEM((125, LANES), jnp.float32),
          pltpu.VMEM_SHARED((N, LANES), jnp.float32),
      ],
  )
  def k(e_hbm, dst_hbm, den_hbm,
        e_v, dst_v, contrib, idxb, zb, den_sp):
    cc = lax.axis_index("c")
    sid = lax.axis_index("s")
    wid = sid * NC + cc
    base = wid * STRIP
    rowb = jnp.arange(LANES, dtype=jnp.int32)
    pltpu.sync_copy(e_hbm.at[wid], e_v)
    pltpu.sync_copy(dst_hbm.at[pl.ds(base, STRIP)], dst_v)

    zv = jnp.zeros((LANES,), jnp.float32)
    for r in range(125):
      zb[r, :] = zv
    for r in range(GE):
      contrib[r, :] = zv
    def zloop(t, _):
      pltpu.sync_copy(zb, den_sp.at[pl.ds(sid * NROW + t * 125, 125)])
      return 0
    lax.fori_loop(0, NROW // 125, zloop, 0)
    plsc.subcore_barrier()

    def group(g, _):
      for bb in range(GROUP):
        b = g * GROUP + bb
        d_idx = dst_v[pl.ds(b * LANES, LANES)]
        idxb[pl.ds(bb * LANES, LANES)] = d_idx
        valid = (b * LANES + rowb) < EV
        for hh in range(Hh):
          ee = jnp.exp(e_v[pl.ds(hh * STRIP + b * LANES, LANES)])
          ee = jnp.where(valid, ee, 0.0)
          plsc.store_scatter(
              contrib,
              [bb * LANES + rowb, jnp.full((LANES,), hh, jnp.int32)], ee)
      pltpu.sync_copy(contrib, den_sp.at[idxb], add=True)
      return 0
    lax.fori_loop(0, NGROUP, group, 0)
    plsc.subcore_barrier()
    pltpu.sync_copy(den_sp.at[pl.ds(sid * NROW, NROW)],
                    den_hbm.at[cc, pl.ds(sid * NROW, NROW)])

  return k


_BLK = 1000


def _tc_feat1(h, W1):
  def body(h_ref, w_ref, o_ref):
    for hh in range(NHEAD):
      o_ref[hh] = jnp.dot(h_ref[...], w_ref[hh],
                          preferred_element_type=jnp.float32)
  return pl.pallas_call(
      body,
      grid=(N // _BLK,),
      in_specs=[pl.BlockSpec((_BLK, D), lambda i: (i, 0)),
                pl.BlockSpec((NHEAD, D, D), lambda i: (0, 0, 0))],
      out_specs=pl.BlockSpec((NHEAD, _BLK, D), lambda i: (0, i, 0)),
      out_shape=jax.ShapeDtypeStruct((NHEAD, N, D), jnp.float32),
  )(h, W1.reshape(D, NHEAD, D).transpose(1, 0, 2))


def _tc_mid(p, den, W2h, rW2h):
  def body(p_ref, d_ref, w_ref, rw_ref, f_ref, r_ref):
    f = jnp.zeros((_BLK, D), jnp.float32)
    r = jnp.zeros((_BLK, D), jnp.float32)
    dsum = d_ref[0] + d_ref[1]
    for hh in range(NHEAD):
      x = (p_ref[0, hh] + p_ref[1, hh]) / (dsum[:, hh:hh + 1] + 1e-9)
      x = jnp.where(x > 0, x, jnp.exp(jnp.minimum(x, 0.0)) - 1.0)
      f = f + jnp.dot(x, w_ref[hh], preferred_element_type=jnp.float32)
      r = r + jnp.dot(x, rw_ref[hh], preferred_element_type=jnp.float32)
    f_ref[...] = f
    r_ref[...] = r
  return pl.pallas_call(
      body,
      grid=(N // _BLK,),
      in_specs=[pl.BlockSpec((NC, NHEAD, _BLK, D), lambda i: (0, 0, i, 0)),
                pl.BlockSpec((NC, _BLK, LANES), lambda i: (0, i, 0)),
                pl.BlockSpec((NHEAD, D, D), lambda i: (0, 0, 0)),
                pl.BlockSpec((NHEAD, D, D), lambda i: (0, 0, 0))],
      out_specs=[pl.BlockSpec((_BLK, D), lambda i: (i, 0)),
                 pl.BlockSpec((_BLK, D), lambda i: (i, 0))],
      out_shape=[jax.ShapeDtypeStruct((N, D), jnp.float32),
                 jax.ShapeDtypeStruct((N, D), jnp.float32)],
  )(p, den, W2h, rW2h)


def _tc_final(p2, den2, res2, Wl, bl2):
  def body(p_ref, d_ref, r_ref, wl_ref, bl_ref, loc_ref, glob_ref, acc_ref):
    i = pl.program_id(0)
    dsum = d_ref[0] + d_ref[1]
    loc = (p_ref[0] + p_ref[1]) / (dsum[:, 0:1] + 1e-9) + r_ref[...]
    loc_ref[...] = loc
    @pl.when(i == 0)
    def _():
      acc_ref[...] = jnp.zeros_like(acc_ref)
    acc_ref[...] += jnp.sum(loc, axis=0, keepdims=True)
    @pl.when(i == pl.num_programs(0) - 1)
    def _():
      glob_ref[...] = (jnp.dot(acc_ref[...] * (1.0 / N), wl_ref[...],
                               preferred_element_type=jnp.float32)
                       + bl_ref[...])
  return pl.pallas_call(
      body,
      grid=(N // _BLK,),
      in_specs=[pl.BlockSpec((NC, _BLK, D), lambda i: (0, i, 0)),
                pl.BlockSpec((NC, _BLK, LANES), lambda i: (0, i, 0)),
                pl.BlockSpec((_BLK, D), lambda i: (i, 0)),
                pl.BlockSpec((D, D), lambda i: (0, 0)),
                pl.BlockSpec((1, D), lambda i: (0, 0))],
      out_specs=[pl.BlockSpec((_BLK, D), lambda i: (i, 0)),
                 pl.BlockSpec((1, D), lambda i: (0, 0))],
      out_shape=[jax.ShapeDtypeStruct((N, D), jnp.float32),
                 jax.ShapeDtypeStruct((1, D), jnp.float32)],
      scratch_shapes=[pltpu.VMEM((1, D), jnp.float32)],
  )(p2, den2, res2, Wl, bl2)


def _sc_layer(feat_hm, src_pad, dst_pad, attn, Hh, dh):
  e, agg = _make_fused(Hh, dh)(feat_hm.reshape(Hh * N, dh), src_pad, dst_pad,
                               attn.reshape(-1))
  den = _make_denom(Hh)(e, dst_pad)
  return agg, den


def kernel(h, edge_index, he, W1, attn1, W2, attn2, resW2, Wl, bl):
  del he
  ei = edge_index.astype(jnp.int32)
  pad2 = jnp.zeros((2, NW, STRIP), jnp.int32)
  pad2 = pad2.at[:, :, :EV].set(ei.reshape(2, NW, EV))
  src_pad = pad2[0].reshape(E_PAD)
  dst_pad = pad2[1].reshape(E_PAD)

  feat1 = _tc_feat1(h, W1)
  p1, den1 = _sc_layer(feat1, src_pad, dst_pad, attn1, NHEAD, D)
  feat2, res2 = _tc_mid(p1, den1, W2.reshape(NHEAD, D, D),
                        resW2.reshape(NHEAD, D, D))
  p2, den2 = _sc_layer(feat2[None], src_pad, dst_pad, attn2, 1, D)
  local_feat, global_feat = _tc_final(p2[:, 0], den2, res2, Wl,
                                      bl.reshape(1, D))
  return (local_feat, global_feat)


# speedup vs baseline: 1.1420x; 1.1420x over previous
import functools

import jax
import jax.numpy as jnp
from jax import lax
from jax.experimental import pallas as pl
from jax.experimental.pallas import tpu as pltpu
from jax.experimental.pallas import tpu_sc as plsc

N = 10000
E = 160000
D = 128
NHEAD = 8

NC = 2
NS = 16
LANES = 16
NW = NC * NS
EV = E // NW
STRIP = 5120
NBATCH = STRIP // LANES
GROUP = 8
NGROUP = NBATCH // GROUP
GE = GROUP * LANES
E_PAD = NW * STRIP
NEG_BIG = -3.0e38

_mesh = lambda: plsc.VectorSubcoreMesh(core_axis_name="c", subcore_axis_name="s")
_params = lambda: pltpu.CompilerParams(use_tc_tiling_on_sc=False,
                                       needs_layout_passes=False)


def _wid():
  return lax.axis_index("s") * NC + lax.axis_index("c")


def _splat(vec, j):
  idx = jnp.full((LANES,), j, dtype=jnp.int32)
  return vec.at[idx].get(mode="promise_in_bounds")


def _global_max(mx_v, Hh):
  ms = []
  for hh in range(Hh):
    v = jnp.full((LANES,), NEG_BIG, jnp.float32)
    for w in range(NW):
      v = jnp.maximum(v, mx_v[w, pl.ds(hh * LANES, LANES)])
    ms.append(jnp.max(v))
  return ms


GROUP_A = 2
GEA = GROUP_A * LANES
NGROUP_A = NBATCH // GROUP_A


def _make_logits(Hh, dh):
  HD = Hh * dh
  NROW = N // NS

  @functools.partial(
      pl.kernel,
      out_type=(jax.ShapeDtypeStruct((NW, Hh * STRIP), jnp.float32),
                jax.ShapeDtypeStruct((NW, Hh * LANES), jnp.float32)),
      mesh=_mesh(),
      compiler_params=_params(),
      scratch_types=[
          pltpu.VMEM((STRIP,), jnp.int32),
          pltpu.VMEM((STRIP,), jnp.int32),
          pltpu.VMEM((HD,), jnp.float32),
          pltpu.VMEM((GEA, dh), jnp.float32),
          pltpu.VMEM((GEA, dh), jnp.float32),
          pltpu.VMEM((GEA, dh), jnp.float32),
          pltpu.VMEM((GEA, dh), jnp.float32),
          pltpu.VMEM((GEA,), jnp.int32),
          pltpu.VMEM((GEA,), jnp.int32),
          pltpu.VMEM((GEA,), jnp.int32),
          pltpu.VMEM((GEA,), jnp.int32),
          pltpu.VMEM((STRIP,), jnp.float32),
          pltpu.VMEM((Hh * LANES,), jnp.float32),
          pltpu.VMEM_SHARED((N, dh), jnp.float32),
          pltpu.SemaphoreType.DMA,
          pltpu.SemaphoreType.DMA,
      ],
  )
  def k(feat_hbm, src_hbm, dst_hbm, attn_hbm, e_hbm, mx_hbm,
        src_v, dst_v, attn_v, fs0, ft0, fs1, ft1,
        gx0, fx0, gx1, fx1, e_h, mx_v, feat_sp, sem0, sem1):
    wid = _wid()
    sid = lax.axis_index("s")
    base = wid * STRIP
    pltpu.sync_copy(src_hbm.at[pl.ds(base, STRIP)], src_v)
    pltpu.sync_copy(dst_hbm.at[pl.ds(base, STRIP)], dst_v)
    pltpu.sync_copy(attn_hbm, attn_v)
    rowb = jnp.arange(LANES, dtype=jnp.int32)
    for hh in range(Hh):
      mx_v[pl.ds(hh * LANES, LANES)] = jnp.full((LANES,), NEG_BIG, jnp.float32)

    def issue(g, fs, ft, gx, fx, sem):
      goff = g * GEA
      for bb in range(GROUP_A):
        sl = pl.ds(bb * LANES, LANES)
        gsl = pl.ds(goff + bb * LANES, LANES)
        gx[sl] = src_v[gsl]
        fx[sl] = dst_v[gsl]
      pltpu.async_copy(feat_sp.at[gx], fs, sem)
      pltpu.async_copy(feat_sp.at[fx], ft, sem)

    def wait(fs, ft, gx, fx, sem):
      pltpu.make_async_copy(feat_sp.at[gx], fs, sem).wait()
      pltpu.make_async_copy(feat_sp.at[fx], ft, sem).wait()

    def compute(hh, g, fs, ft):
      for bb in range(GROUP_A):
        rowv = bb * LANES + rowb
        def chunk(c2, acc):
          dbase = c2 * LANES
          a_chunk = attn_v[pl.ds(hh * dh + dbase, LANES)]
          for j in range(LANES):
            colv = jnp.full((LANES,), dbase + j, dtype=jnp.int32)
            z = (plsc.load_gather(fs, [rowv, colv]) +
                 plsc.load_gather(ft, [rowv, colv]))
            z = jnp.maximum(z, 0.2 * z)
            acc = acc + z * _splat(a_chunk, j)
          return acc
        acc = lax.fori_loop(0, dh // LANES, chunk,
                            jnp.zeros((LANES,), jnp.float32))
        boff = g * GEA + bb * LANES
        e_h[pl.ds(boff, LANES)] = acc
        accm = jnp.where(boff + rowb < EV, acc, NEG_BIG)
        moff = hh * LANES
        mx_v[pl.ds(moff, LANES)] = jnp.maximum(mx_v[pl.ds(moff, LANES)], accm)

    def head(hh, _):
      plsc.subcore_barrier()
      pltpu.sync_copy(feat_hbm.at[hh, pl.ds(sid * NROW, NROW)],
                      feat_sp.at[pl.ds(sid * NROW, NROW)])
      plsc.subcore_barrier()
      issue(0, fs0, ft0, gx0, fx0, sem0)
      issue(1, fs1, ft1, gx1, fx1, sem1)

      def outer(i, _o):
        g0 = 2 * i
        wait(fs0, ft0, gx0, fx0, sem0)
        compute(hh, g0, fs0, ft0)
        @pl.when(g0 + 2 < NGROUP_A)
        def _():
          issue(g0 + 2, fs0, ft0, gx0, fx0, sem0)
        g1 = g0 + 1
        wait(fs1, ft1, gx1, fx1, sem1)
        compute(hh, g1, fs1, ft1)
        @pl.when(g1 + 2 < NGROUP_A)
        def _():
          issue(g1 + 2, fs1, ft1, gx1, fx1, sem1)
        return 0
      lax.fori_loop(0, NGROUP_A // 2, outer, 0)
      pltpu.sync_copy(e_h, e_hbm.at[wid, pl.ds(hh * STRIP, STRIP)])
      return 0
    lax.fori_loop(0, Hh, head, 0)
    pltpu.sync_copy(mx_v, mx_hbm.at[wid])

  return k


def _make_denom(Hh):
  NROW = N // NS

  @functools.partial(
      pl.kernel,
      out_type=jax.ShapeDtypeStruct((NC, N, LANES), jnp.float32),
      mesh=_mesh(),
      compiler_params=_params(),
      scratch_types=[
          pltpu.VMEM((Hh * STRIP,), jnp.float32),
          pltpu.VMEM((STRIP,), jnp.int32),
          pltpu.VMEM((NW, Hh * LANES), jnp.float32),
          pltpu.VMEM((GE, LANES), jnp.float32),
          pltpu.VMEM((GE,), jnp.int32),
          pltpu.VMEM((125, LANES), jnp.float32),
          pltpu.VMEM_SHARED((N, LANES), jnp.float32),
      ],
  )
  def k(e_hbm, mx_hbm, dst_hbm, den_hbm,
        e_v, dst_v, mx_v, contrib, idxb, zb, den_sp):
    cc = lax.axis_index("c")
    sid = lax.axis_index("s")
    wid = sid * NC + cc
    base = wid * STRIP
    rowb = jnp.arange(LANES, dtype=jnp.int32)
    pltpu.sync_copy(e_hbm.at[wid], e_v)
    pltpu.sync_copy(dst_hbm.at[pl.ds(base, STRIP)], dst_v)
    pltpu.sync_copy(mx_hbm, mx_v)
    m = _global_max(mx_v, Hh)

    zv = jnp.zeros((LANES,), jnp.float32)
    for r in range(125):
      zb[r, :] = zv
    for r in range(GE):
      contrib[r, :] = zv
    def zloop(t, _):
      pltpu.sync_copy(zb, den_sp.at[pl.ds(sid * NROW + t * 125, 125)])
      return 0
    lax.fori_loop(0, NROW // 125, zloop, 0)
    plsc.subcore_barrier()

    def group(g, _):
      for bb in range(GROUP):
        b = g * GROUP + bb
        d_idx = dst_v[pl.ds(b * LANES, LANES)]
        idxb[pl.ds(bb * LANES, LANES)] = d_idx
        valid = (b * LANES + rowb) < EV
        for hh in range(Hh):
          ee = jnp.exp(e_v[pl.ds(hh * STRIP + b * LANES, LANES)] - m[hh])
          ee = jnp.where(valid, ee, 0.0)
          plsc.store_scatter(
              contrib,
              [bb * LANES + rowb, jnp.full((LANES,), hh, jnp.int32)], ee)
      pltpu.sync_copy(contrib, den_sp.at[idxb], add=True)
      return 0
    lax.fori_loop(0, NGROUP, group, 0)
    plsc.subcore_barrier()
    pltpu.sync_copy(den_sp.at[pl.ds(sid * NROW, NROW)],
                    den_hbm.at[cc, pl.ds(sid * NROW, NROW)])

  return k


GROUP_D = 5
GED = GROUP_D * LANES
NGROUP_D = NBATCH // GROUP_D


def _make_aggregate(Hh, dh):
  NROW = N // NS

  @functools.partial(
      pl.kernel,
      out_type=jax.ShapeDtypeStruct((NC, Hh, N, dh), jnp.float32),
      mesh=_mesh(),
      compiler_params=_params(),
      scratch_types=[
          pltpu.VMEM((STRIP,), jnp.int32),
          pltpu.VMEM((STRIP,), jnp.int32),
          pltpu.VMEM((STRIP,), jnp.float32),
          pltpu.VMEM((NW, Hh * LANES), jnp.float32),
          pltpu.VMEM((GED,), jnp.int32),
          pltpu.VMEM((GED,), jnp.int32),
          pltpu.VMEM((GED,), jnp.int32),
          pltpu.VMEM((GED,), jnp.int32),
          pltpu.VMEM((GED, dh), jnp.float32),
          pltpu.VMEM((GED, dh), jnp.float32),
          pltpu.VMEM((125, LANES), jnp.float32),
          pltpu.VMEM_SHARED((N, dh), jnp.float32),
          pltpu.SemaphoreType.DMA,
          pltpu.SemaphoreType.DMA,
          pltpu.SemaphoreType.DMA,
          pltpu.SemaphoreType.DMA,
      ],
  )
  def k(feat_hbm, e_hbm, mx_hbm, src_hbm, dst_hbm, out_hbm,
        src_v, dst_v, e_h, mx_v, gx0, sx0, gx1, sx1, rows0, rows1, zb,
        acc_sp, semg0, semg1, sems0, sems1):
    cc = lax.axis_index("c")
    sid = lax.axis_index("s")
    wid = sid * NC + cc
    base = wid * STRIP
    rowb = jnp.arange(LANES, dtype=jnp.int32)
    pltpu.sync_copy(src_hbm.at[pl.ds(base, STRIP)], src_v)
    pltpu.sync_copy(dst_hbm.at[pl.ds(base, STRIP)], dst_v)
    pltpu.sync_copy(mx_hbm, mx_v)
    ms = _global_max(mx_v, Hh)
    m_v = jnp.zeros((LANES,), jnp.float32)
    for hh in range(Hh):
      m_v = jnp.where(rowb == hh, ms[hh], m_v)
    zv = jnp.zeros((LANES,), jnp.float32)
    for r in range(125):
      zb[r, :] = zv

    def issue(hh, g, rows, gx, sx, semg):
      goff = g * GED
      for bb in range(GROUP_D):
        sl = pl.ds(bb * LANES, LANES)
        gsl = pl.ds(goff + bb * LANES, LANES)
        gx[sl] = src_v[gsl] + hh * N
        sx[sl] = dst_v[gsl]
      pltpu.async_copy(feat_hbm.at[gx], rows, semg)

    def scale(hh, g, rows, m_h):
      for bb in range(GROUP_D):
        boff = g * GED + bb * LANES
        ee = jnp.exp(e_h[pl.ds(boff, LANES)] - m_h)
        ee = jnp.where((boff + rowb) < EV, ee, 0.0)
        for j in range(LANES):
          asp = _splat(ee, j)
          r = bb * LANES + j
          for c2 in range(dh // LANES):
            sl = pl.ds(c2 * LANES, LANES)
            rows[r, sl] = rows[r, sl] * asp

    def head(hh, _):
      def zloop(t, _z):
        r0 = sid * NROW + (t // (dh // LANES)) * 125
        c0 = (t % (dh // LANES)) * LANES
        pltpu.sync_copy(zb, acc_sp.at[pl.ds(r0, 125), pl.ds(c0, LANES)])
        return 0
      lax.fori_loop(0, (NROW // 125) * (dh // LANES), zloop, 0)
      pltpu.sync_copy(
          e_hbm.at[wid, pl.ds(hh * STRIP, STRIP)], e_h)
      m_h = _splat(m_v, hh)
      plsc.subcore_barrier()

      issue(hh, 0, rows0, gx0, sx0, semg0)
      issue(hh, 1, rows1, gx1, sx1, semg1)

      def outer(i, _o):
        g0 = 2 * i
        pltpu.make_async_copy(feat_hbm.at[gx0], rows0, semg0).wait()
        scale(hh, g0, rows0, m_h)
        pltpu.async_copy(rows0, acc_sp.at[sx0], sems0, add=True)
        g1 = g0 + 1
        pltpu.make_async_copy(feat_hbm.at[gx1], rows1, semg1).wait()
        scale(hh, g1, rows1, m_h)
        pltpu.async_copy(rows1, acc_sp.at[sx1], sems1, add=True)
        pltpu.make_async_copy(rows0, acc_sp.at[sx0], sems0).wait()
        @pl.when(g0 + 2 < NGROUP_D)
        def _():
          issue(hh, g0 + 2, rows0, gx0, sx0, semg0)
        pltpu.make_async_copy(rows1, acc_sp.at[sx1], sems1).wait()
        @pl.when(g1 + 2 < NGROUP_D)
        def _():
          issue(hh, g1 + 2, rows1, gx1, sx1, semg1)
        return 0
      lax.fori_loop(0, NGROUP_D // 2, outer, 0)
      plsc.subcore_barrier()
      pltpu.sync_copy(acc_sp.at[pl.ds(sid * NROW, NROW)],
                      out_hbm.at[cc, hh, pl.ds(sid * NROW, NROW)])
      plsc.subcore_barrier()
      return 0
    lax.fori_loop(0, Hh, head, 0)

  return k


_BLK = 1000


def _tc_feat1(h, W1):
  def body(h_ref, w_ref, o_ref):
    for hh in range(NHEAD):
      o_ref[hh] = jnp.dot(h_ref[...], w_ref[hh],
                          preferred_element_type=jnp.float32)
  return pl.pallas_call(
      body,
      grid=(N // _BLK,),
      in_specs=[pl.BlockSpec((_BLK, D), lambda i: (i, 0)),
                pl.BlockSpec((NHEAD, D, D), lambda i: (0, 0, 0))],
      out_specs=pl.BlockSpec((NHEAD, _BLK, D), lambda i: (0, i, 0)),
      out_shape=jax.ShapeDtypeStruct((NHEAD, N, D), jnp.float32),
  )(h, W1.reshape(D, NHEAD, D).transpose(1, 0, 2))


def _tc_mid(p, den, W2h, rW2h):
  def body(p_ref, d_ref, w_ref, rw_ref, f_ref, r_ref):
    f = jnp.zeros((_BLK, D), jnp.float32)
    r = jnp.zeros((_BLK, D), jnp.float32)
    dsum = d_ref[0] + d_ref[1]
    for hh in range(NHEAD):
      x = (p_ref[0, hh] + p_ref[1, hh]) / (dsum[:, hh:hh + 1] + 1e-9)
      x = jnp.where(x > 0, x, jnp.exp(jnp.minimum(x, 0.0)) - 1.0)
      f = f + jnp.dot(x, w_ref[hh], preferred_element_type=jnp.float32)
      r = r + jnp.dot(x, rw_ref[hh], preferred_element_type=jnp.float32)
    f_ref[...] = f
    r_ref[...] = r
  return pl.pallas_call(
      body,
      grid=(N // _BLK,),
      in_specs=[pl.BlockSpec((NC, NHEAD, _BLK, D), lambda i: (0, 0, i, 0)),
                pl.BlockSpec((NC, _BLK, LANES), lambda i: (0, i, 0)),
                pl.BlockSpec((NHEAD, D, D), lambda i: (0, 0, 0)),
                pl.BlockSpec((NHEAD, D, D), lambda i: (0, 0, 0))],
      out_specs=[pl.BlockSpec((_BLK, D), lambda i: (i, 0)),
                 pl.BlockSpec((_BLK, D), lambda i: (i, 0))],
      out_shape=[jax.ShapeDtypeStruct((N, D), jnp.float32),
                 jax.ShapeDtypeStruct((N, D), jnp.float32)],
  )(p, den, W2h, rW2h)


def _tc_final(p2, den2, res2, Wl, bl2):
  def body(p_ref, d_ref, r_ref, wl_ref, bl_ref, loc_ref, glob_ref, acc_ref):
    i = pl.program_id(0)
    dsum = d_ref[0] + d_ref[1]
    loc = (p_ref[0] + p_ref[1]) / (dsum[:, 0:1] + 1e-9) + r_ref[...]
    loc_ref[...] = loc
    @pl.when(i == 0)
    def _():
      acc_ref[...] = jnp.zeros_like(acc_ref)
    acc_ref[...] += jnp.sum(loc, axis=0, keepdims=True)
    @pl.when(i == pl.num_programs(0) - 1)
    def _():
      glob_ref[...] = (jnp.dot(acc_ref[...] * (1.0 / N), wl_ref[...],
                               preferred_element_type=jnp.float32)
                       + bl_ref[...])
  return pl.pallas_call(
      body,
      grid=(N // _BLK,),
      in_specs=[pl.BlockSpec((NC, _BLK, D), lambda i: (0, i, 0)),
                pl.BlockSpec((NC, _BLK, LANES), lambda i: (0, i, 0)),
                pl.BlockSpec((_BLK, D), lambda i: (i, 0)),
                pl.BlockSpec((D, D), lambda i: (0, 0)),
                pl.BlockSpec((1, D), lambda i: (0, 0))],
      out_specs=[pl.BlockSpec((_BLK, D), lambda i: (i, 0)),
                 pl.BlockSpec((1, D), lambda i: (0, 0))],
      out_shape=[jax.ShapeDtypeStruct((N, D), jnp.float32),
                 jax.ShapeDtypeStruct((1, D), jnp.float32)],
      scratch_shapes=[pltpu.VMEM((1, D), jnp.float32)],
  )(p2, den2, res2, Wl, bl2)


def _sc_layer(feat_hm, src_pad, dst_pad, attn, Hh, dh):
  e, mx = _make_logits(Hh, dh)(feat_hm, src_pad, dst_pad, attn.reshape(-1))
  den = _make_denom(Hh)(e, mx, dst_pad)
  agg = _make_aggregate(Hh, dh)(feat_hm.reshape(Hh * N, dh), e, mx,
                                src_pad, dst_pad)
  return agg, den


def kernel(h, edge_index, he, W1, attn1, W2, attn2, resW2, Wl, bl):
  del he
  ei = edge_index.astype(jnp.int32)
  pad2 = jnp.zeros((2, NW, STRIP), jnp.int32)
  pad2 = pad2.at[:, :, :EV].set(ei.reshape(2, NW, EV))
  src_pad = pad2[0].reshape(E_PAD)
  dst_pad = pad2[1].reshape(E_PAD)

  feat1 = _tc_feat1(h, W1)
  p1, den1 = _sc_layer(feat1, src_pad, dst_pad, attn1, NHEAD, D)
  feat2, res2 = _tc_mid(p1, den1, W2.reshape(NHEAD, D, D),
                        resW2.reshape(NHEAD, D, D))
  p2, den2 = _sc_layer(feat2[None], src_pad, dst_pad, attn2, 1, D)
  local_feat, global_feat = _tc_final(p2[:, 0], den2, res2, Wl,
                                      bl.reshape(1, D))
  return (local_feat, global_feat)
